# baseline (speedup 1.0000x reference)
import functools

import jax
import jax.numpy as jnp
from jax.experimental import pallas as pl
from jax.experimental.pallas import tpu as pltpu

N_COMPONENTS = 1024
EMBEDDING_DIM = 64
BETA = 0.25


def _vq_block(x_ref, cb_ref, q_ref, sc_ref, loss_ref):
    x = x_ref[...].reshape(-1, EMBEDDING_DIM)
    cb = cb_ref[...]
    c2 = jnp.sum(cb * cb, axis=0, keepdims=True)
    sim2 = jnp.dot(x, -2.0 * cb, preferred_element_type=jnp.float32,
                   precision=jax.lax.Precision.HIGHEST)
    x2 = jnp.sum(x * x, axis=1, keepdims=True)
    d = (c2 + sim2) + x2
    r = 1.0 / d
    inv = r * r
    rows = jnp.sum(inv, axis=1, keepdims=True)
    imax = jnp.max(inv, axis=1, keepdims=True)
    sc_ref[...] = inv * (1.0 / rows)
    onehot = (inv == imax).astype(jnp.float32)
    q = jax.lax.dot_general(onehot, cb, (((1,), (1,)), ((), ())),
                            preferred_element_type=jnp.float32)
    q_ref[...] = q.reshape(q_ref.shape)
    loss_ref[...] = jnp.sum(jax.lax.rsqrt(imax)).reshape(1, 1, 1)


@jax.jit
def kernel(x, codebook):
    b, t, _ = x.shape
    n = b * t
    q, sc, loss = pl.pallas_call(
        _vq_block,
        grid=(b // 2,),
        in_specs=[
            pl.BlockSpec((2, t, EMBEDDING_DIM), lambda i: (i, 0, 0)),
            pl.BlockSpec((EMBEDDING_DIM, N_COMPONENTS), lambda i: (0, 0)),
        ],
        out_specs=[
            pl.BlockSpec((2, t, EMBEDDING_DIM), lambda i: (i, 0, 0)),
            pl.BlockSpec((2 * t, N_COMPONENTS), lambda i: (i, 0)),
            pl.BlockSpec((1, 1, 1), lambda i: (i, 0, 0)),
        ],
        out_shape=[
            jax.ShapeDtypeStruct((b, t, EMBEDDING_DIM), jnp.float32),
            jax.ShapeDtypeStruct((n, N_COMPONENTS), jnp.float32),
            jax.ShapeDtypeStruct((b // 2, 1, 1), jnp.float32),
        ],
        compiler_params=pltpu.CompilerParams(
            dimension_semantics=("parallel",)),
    )(x, codebook)
    vq_loss = (1.0 + BETA) * jnp.sum(loss) / (n * EMBEDDING_DIM)
    return q, sc, vq_loss


# speedup vs baseline: 1.8872x; 1.8872x over previous
import functools

import jax
import jax.numpy as jnp
from jax.experimental import pallas as pl
from jax.experimental.pallas import tpu as pltpu

N_COMPONENTS = 1024
EMBEDDING_DIM = 64
BETA = 0.25


def _vq_block(x_ref, cb_ref, q_ref, sc_ref, loss_ref):
    cb = cb_ref[...]
    c2 = jnp.sum(cb * cb, axis=0, keepdims=True)
    cbm2 = -2.0 * cb
    t = x_ref.shape[1]
    loss = jnp.zeros((1, 1, 1), jnp.float32)
    for h in range(x_ref.shape[0]):
        x = x_ref[h]
        sim2 = jnp.dot(x, cbm2, preferred_element_type=jnp.float32)
        x2 = jnp.sum(x * x, axis=1, keepdims=True)
        d = (c2 + sim2) + x2
        r = 1.0 / d
        inv = r * r
        rows = jnp.sum(inv, axis=1, keepdims=True)
        imax = jnp.max(inv, axis=1, keepdims=True)
        sc_ref[pl.ds(h * t, t), :] = inv * (1.0 / rows)
        onehot = (inv == imax).astype(jnp.float32)
        q = jax.lax.dot_general(onehot, cb, (((1,), (1,)), ((), ())),
                                preferred_element_type=jnp.float32)
        q_ref[h] = q
        loss = loss + jnp.sum(jax.lax.rsqrt(imax)).reshape(1, 1, 1)
    loss_ref[...] = loss


@jax.jit
def kernel(x, codebook):
    b, t, _ = x.shape
    n = b * t
    q, sc, loss = pl.pallas_call(
        _vq_block,
        grid=(b // 2,),
        in_specs=[
            pl.BlockSpec((2, t, EMBEDDING_DIM), lambda i: (i, 0, 0)),
            pl.BlockSpec((EMBEDDING_DIM, N_COMPONENTS), lambda i: (0, 0)),
        ],
        out_specs=[
            pl.BlockSpec((2, t, EMBEDDING_DIM), lambda i: (i, 0, 0)),
            pl.BlockSpec((2 * t, N_COMPONENTS), lambda i: (i, 0)),
            pl.BlockSpec((1, 1, 1), lambda i: (i, 0, 0)),
        ],
        out_shape=[
            jax.ShapeDtypeStruct((b, t, EMBEDDING_DIM), jnp.float32),
            jax.ShapeDtypeStruct((n, N_COMPONENTS), jnp.float32),
            jax.ShapeDtypeStruct((b // 2, 1, 1), jnp.float32),
        ],
        compiler_params=pltpu.CompilerParams(
            dimension_semantics=("parallel",)),
    )(x, codebook)
    vq_loss = (1.0 + BETA) * jnp.sum(loss) / (n * EMBEDDING_DIM)
    return q, sc, vq_loss


# speedup vs baseline: 2.0278x; 1.0745x over previous
import functools

import jax
import jax.numpy as jnp
from jax.experimental import pallas as pl
from jax.experimental.pallas import tpu as pltpu

N_COMPONENTS = 1024
EMBEDDING_DIM = 64
BETA = 0.25


def _vq_block(x_ref, cb_ref, q_ref, sc_ref, loss_ref):
    cb = cb_ref[...]
    c2 = jnp.sum(cb * cb, axis=0, keepdims=True)
    cbm2 = -2.0 * cb
    t = x_ref.shape[1]
    loss = jnp.zeros((1, 1, 1), jnp.float32)
    for h in range(x_ref.shape[0]):
        x = x_ref[h]
        sim2 = jnp.dot(x, cbm2, preferred_element_type=jnp.float32)
        x2 = jnp.sum(x * x, axis=1, keepdims=True)
        d = (c2 + sim2) + x2
        r = 1.0 / d
        inv = r * r
        rows = jnp.sum(inv, axis=1, keepdims=True)
        imax = jnp.max(inv, axis=1, keepdims=True)
        sc_ref[pl.ds(h * t, t), :] = inv * (1.0 / rows)
        onehot = (inv == imax).astype(jnp.float32)
        q = jax.lax.dot_general(onehot, cb, (((1,), (1,)), ((), ())),
                                preferred_element_type=jnp.float32)
        q_ref[h] = q
        loss = loss + jnp.sum(jax.lax.rsqrt(imax)).reshape(1, 1, 1)
    loss_ref[...] = loss


@jax.jit
def kernel(x, codebook):
    b, t, _ = x.shape
    n = b * t
    q, sc, loss = pl.pallas_call(
        _vq_block,
        grid=(b // 4,),
        in_specs=[
            pl.BlockSpec((4, t, EMBEDDING_DIM), lambda i: (i, 0, 0)),
            pl.BlockSpec((EMBEDDING_DIM, N_COMPONENTS), lambda i: (0, 0)),
        ],
        out_specs=[
            pl.BlockSpec((4, t, EMBEDDING_DIM), lambda i: (i, 0, 0)),
            pl.BlockSpec((4 * t, N_COMPONENTS), lambda i: (i, 0)),
            pl.BlockSpec((1, 1, 1), lambda i: (i, 0, 0)),
        ],
        out_shape=[
            jax.ShapeDtypeStruct((b, t, EMBEDDING_DIM), jnp.float32),
            jax.ShapeDtypeStruct((n, N_COMPONENTS), jnp.float32),
            jax.ShapeDtypeStruct((b // 4, 1, 1), jnp.float32),
        ],
        compiler_params=pltpu.CompilerParams(
            dimension_semantics=("parallel",)),
    )(x, codebook)
    vq_loss = (1.0 + BETA) * jnp.sum(loss) / (n * EMBEDDING_DIM)
    return q, sc, vq_loss
